# Initial kernel scaffold; baseline (speedup 1.0000x reference)
import jax
import jax.numpy as jnp
from jax.experimental import pallas as pl


def kernel(x, edge_index0, edge_index1, edge_index2, Wl0, Wr0, b0, Wl1, Wr1, b1, Wl2, Wr2, b2):
    raise NotImplementedError("write your pallas kernel here")


# speedup vs baseline: 1.3801x; 1.3801x over previous
import jax
import jax.numpy as jnp
from jax import lax
from jax.experimental import pallas as pl
from jax.experimental.pallas import tpu as pltpu
from jax.experimental.pallas import tpu_sc as plsc

NC = 2
NS = 16
NW = NC * NS
K = 96
D = 128


def _make_sc_gather(e_pad):
    et = e_pad // NW
    nblocks = et // K

    mesh = plsc.VectorSubcoreMesh(
        core_axis_name="c", subcore_axis_name="s",
        num_cores=NC, num_subcores=NS)

    def body(idx_hbm, h_hbm, out_hbm, idx_v, rows_v, sem):
        cid = lax.axis_index("c")
        sid = lax.axis_index("s")
        wid = sid * NC + cid

        def block(b, c):
            base = wid * et + b * K
            pltpu.sync_copy(idx_hbm.at[pl.ds(base, K)], idx_v)
            pltpu.async_copy(h_hbm.at[idx_v], rows_v, sem).wait()
            pltpu.sync_copy(rows_v, out_hbm.at[pl.ds(base, K)])
            return c
        lax.fori_loop(0, nblocks, block, 0)

    return pl.kernel(
        body,
        out_type=jax.ShapeDtypeStruct((e_pad, D), jnp.float32),
        mesh=mesh,
        scratch_types=[
            pltpu.VMEM((K,), jnp.int32),
            pltpu.VMEM((K, D), jnp.float32),
            pltpu.SemaphoreType.DMA,
        ],
    )


def _make_tc_dense(n_dst, r, relu):
    def body(sums_ref, cnts_ref, h_ref, wl_ref, wr_ref, b_ref, o_ref):
        mean = sums_ref[...] / cnts_ref[...]
        acc = jnp.dot(mean, wl_ref[...],
                      preferred_element_type=jnp.float32,
                      precision=lax.Precision.HIGHEST)
        acc = acc + jnp.dot(h_ref[...], wr_ref[...],
                            preferred_element_type=jnp.float32,
                            precision=lax.Precision.HIGHEST)
        acc = acc + b_ref[...]
        if relu:
            acc = jnp.maximum(acc, 0.0)
        o_ref[...] = acc

    return pl.pallas_call(
        body,
        grid=(n_dst // r,),
        in_specs=[
            pl.BlockSpec((r, D), lambda i: (i, 0)),
            pl.BlockSpec((r, D), lambda i: (i, 0)),
            pl.BlockSpec((r, D), lambda i: (i, 0)),
            pl.BlockSpec((D, D), lambda i: (0, 0)),
            pl.BlockSpec((D, D), lambda i: (0, 0)),
            pl.BlockSpec((1, D), lambda i: (0, 0)),
        ],
        out_specs=pl.BlockSpec((r, D), lambda i: (i, 0)),
        out_shape=jax.ShapeDtypeStruct((n_dst, D), jnp.float32),
    )


_LAYERS = [
    (20000, 320000, 322560, 400, True),
    (8000, 128000, 129024, 400, True),
    (2048, 32768, 33792, 256, False),
]


@jax.jit
def kernel(x, edge_index0, edge_index1, edge_index2,
           Wl0, Wr0, b0, Wl1, Wr1, b1, Wl2, Wr2, b2):
    edges = [edge_index0, edge_index1, edge_index2]
    params = [(Wl0, Wr0, b0), (Wl1, Wr1, b1), (Wl2, Wr2, b2)]
    h = x
    for li, (n_dst, e, e_pad, r, relu) in enumerate(_LAYERS):
        src, dst = edges[li][0], edges[li][1]
        if e_pad != e:
            src = jnp.concatenate(
                [src, jnp.zeros((e_pad - e,), jnp.int32)])
        msgs = _make_sc_gather(e_pad)(src, h)[:e]
        sums = jax.ops.segment_sum(msgs, dst, num_segments=n_dst)
        cnt = jax.ops.segment_sum(jnp.ones((e,), jnp.float32), dst,
                                  num_segments=n_dst)
        cntb = jnp.broadcast_to(jnp.maximum(cnt, 1.0)[:, None],
                                (n_dst, D))
        wl, wr, b = params[li]
        h = _make_tc_dense(n_dst, r, relu)(
            sums, cntb, h[:n_dst], wl, wr, b.reshape(1, D))
    return h
